# baseline (speedup 1.0000x reference)
import functools

import jax
import jax.numpy as jnp
from jax import lax
from jax.experimental import pallas as pl
from jax.experimental.pallas import tpu as pltpu
from jax.experimental.pallas import tpu_sc as plsc


def kernel(x, pos_embed):
    b, s, d = x.shape
    info = plsc.get_sparse_core_info()
    nw = info.num_cores * info.num_subcores
    rows = s // nw
    mesh = plsc.VectorSubcoreMesh(core_axis_name="c", subcore_axis_name="s")

    @functools.partial(
        pl.kernel,
        mesh=mesh,
        out_type=jax.ShapeDtypeStruct((b, s, 2 * d), x.dtype),
    )
    def k(x_hbm, pe_hbm, out_hbm):
        wid = lax.axis_index("s") * info.num_cores + lax.axis_index("c")
        s0 = wid * rows
        for bi in range(b):
            pltpu.sync_copy(
                x_hbm.at[bi, pl.ds(s0, rows), :],
                out_hbm.at[bi, pl.ds(s0, rows), pl.ds(0, d)],
            )
            pltpu.sync_copy(
                pe_hbm.at[pl.ds(s0, rows), :],
                out_hbm.at[bi, pl.ds(s0, rows), pl.ds(d, d)],
            )

    return k(x, pos_embed)


# speedup vs baseline: 46.7612x; 46.7612x over previous
import functools

import jax
import jax.numpy as jnp
from jax import lax
from jax.experimental import pallas as pl
from jax.experimental.pallas import tpu as pltpu
from jax.experimental.pallas import tpu_sc as plsc


_NC, _NS = 2, 16
_CHUNK = 32


def kernel(x, pos_embed):
    b, s, d = x.shape
    nw = _NC * _NS
    rows = s // nw
    n = _CHUNK
    mesh = plsc.VectorSubcoreMesh(core_axis_name="c", subcore_axis_name="s")

    @functools.partial(
        pl.kernel,
        mesh=mesh,
        out_type=jax.ShapeDtypeStruct((b, s, 2 * d), x.dtype),
        scratch_types=[
            pltpu.VMEM((n, d), jnp.float32),
            pltpu.VMEM((n, d), jnp.float32),
            pltpu.SemaphoreType.DMA,
            pltpu.SemaphoreType.DMA,
            pltpu.SemaphoreType.DMA,
            pltpu.SemaphoreType.DMA,
        ],
    )
    def k(x_hbm, pe_hbm, out_hbm, buf0, buf1, si0, si1, so0, so1):
        wid = lax.axis_index("s") * _NC + lax.axis_index("c")
        s0 = wid * rows
        bufs = (buf0, buf1)
        sin = (si0, si1)
        sout = (so0, so1)
        pending = [[], []]

        def use_slot(slot, gather_src, scatter_dsts):
            for h in pending[slot]:
                h.wait()
            pltpu.async_copy(gather_src, bufs[slot], sin[slot]).wait()
            pending[slot] = [
                pltpu.async_copy(bufs[slot], dst, sout[slot])
                for dst in scatter_dsts
            ]

        it = 0
        for bi in range(b):
            for c in range(rows // n):
                r0 = s0 + c * n
                use_slot(
                    it % 2,
                    x_hbm.at[bi, pl.ds(r0, n), :],
                    [out_hbm.at[bi, pl.ds(r0, n), pl.ds(0, d)]],
                )
                it += 1
        for c in range(rows // n):
            r0 = s0 + c * n
            use_slot(
                it % 2,
                pe_hbm.at[pl.ds(r0, n), :],
                [out_hbm.at[bi, pl.ds(r0, n), pl.ds(d, d)] for bi in range(b)],
            )
            it += 1
        for slot in (0, 1):
            for h in pending[slot]:
                h.wait()

    return k(x, pos_embed)
